# Initial kernel scaffold; baseline (speedup 1.0000x reference)
import functools

import jax
import jax.numpy as jnp
from jax import lax
from jax.experimental import pallas as pl
from jax.experimental.pallas import tpu as pltpu
from jax.experimental.pallas import tpu_sc as plsc

_NC = 2
_NS = 16
_NW = _NC * _NS
_C = 80


def _sc_scatter_sum(table, src, dst, zeros):
    n, d = table.shape
    e = src.shape[0]
    assert e % _NW == 0
    epw = e // _NW
    assert epw % _C == 0
    nchunk = epw // _C
    assert n % _NS == 0
    rpt = n // _NS

    mesh = plsc.VectorSubcoreMesh(core_axis_name="c", subcore_axis_name="s")

    @functools.partial(
        pl.kernel,
        mesh=mesh,
        out_type=jax.ShapeDtypeStruct((_NC * n, d), jnp.float32),
        scratch_types=[
            pltpu.VMEM((_C,), jnp.int32),
            pltpu.VMEM((_C,), jnp.int32),
            pltpu.VMEM((_C, d), jnp.float32),
            pltpu.VMEM_SHARED((n, d), jnp.float32),
            pltpu.SemaphoreType.DMA,
        ],
    )
    def k(table_hbm, src_hbm, dst_hbm, zeros_hbm, out_hbm,
          srcv, dstv, rows, acc, sem):
        c = lax.axis_index("c")
        s = lax.axis_index("s")
        w = c * _NS + s

        pltpu.sync_copy(zeros_hbm.at[pl.ds(s * rpt, rpt)],
                        acc.at[pl.ds(s * rpt, rpt)])
        plsc.subcore_barrier()

        @pl.loop(0, nchunk)
        def _(i):
            base = w * epw + i * _C
            pltpu.sync_copy(src_hbm.at[pl.ds(base, _C)], srcv)
            pltpu.sync_copy(dst_hbm.at[pl.ds(base, _C)], dstv)
            pltpu.async_copy(table_hbm.at[srcv], rows, sem).wait()
            pltpu.sync_copy(rows, acc.at[dstv], add=True)

        plsc.subcore_barrier()
        pltpu.sync_copy(acc.at[pl.ds(s * rpt, rpt)],
                        out_hbm.at[pl.ds(c * n + s * rpt, rpt)])

    return k(table, src, dst, zeros)


def _tc_layer1(x, w1, d0, d1):
    n = x.shape[0]
    dh = w1.shape[1]

    def body(x_ref, w_ref, d0_ref, d1_ref, y_ref, dinv_ref):
        deg = d0_ref[...] + d1_ref[...] + 1.0
        dinv = lax.rsqrt(deg)
        dinv_ref[...] = dinv
        xw = jnp.dot(x_ref[...], w_ref[...],
                     preferred_element_type=jnp.float32)
        y_ref[...] = xw * dinv

    return pl.pallas_call(
        body,
        out_shape=(
            jax.ShapeDtypeStruct((n, dh), jnp.float32),
            jax.ShapeDtypeStruct((n, 1), jnp.float32),
        ),
    )(x, w1, d0, d1)


def _tc_mid(a, y1, dinv, b1):
    n, dh = y1.shape

    def body(a_ref, y_ref, dinv_ref, b_ref, u_ref):
        total = a_ref[:n] + a_ref[n:] + y_ref[...]
        h = jnp.maximum(dinv_ref[...] * total + b_ref[...], 0.0)
        u_ref[...] = dinv_ref[...] * h

    return pl.pallas_call(
        body,
        out_shape=jax.ShapeDtypeStruct((n, dh), jnp.float32),
    )(a, y1, dinv, b1)


def _tc_out(a, u, dinv, w2p, b2p):
    n, dh = u.shape
    dp = w2p.shape[1]

    def body(a_ref, u_ref, dinv_ref, w_ref, b_ref, o_ref):
        m = dinv_ref[...] * (a_ref[:n] + a_ref[n:] + u_ref[...])
        o_ref[...] = jnp.dot(m, w_ref[...],
                             preferred_element_type=jnp.float32) + b_ref[...]

    return pl.pallas_call(
        body,
        out_shape=jax.ShapeDtypeStruct((n, dp), jnp.float32),
    )(a, u, dinv, w2p, b2p)


def kernel(x, edge_index, W1, b1, W2, b2):
    n = x.shape[0]
    dh = W1.shape[1]
    dout = W2.shape[1]
    src = edge_index[0]
    dst = edge_index[1]

    zeros_h = jnp.zeros((n, dh), jnp.float32)
    zeros_d = jnp.zeros((n, 16), jnp.float32)
    ones_d = jnp.ones((n, 16), jnp.float32)

    degp = _sc_scatter_sum(ones_d, src, dst, zeros_d)
    d0 = degp[:n, 0:1]
    d1 = degp[n:, 0:1]

    y1, dinv = _tc_layer1(x, W1, d0, d1)
    a1 = _sc_scatter_sum(y1, src, dst, zeros_h)
    u = _tc_mid(a1, y1, dinv, b1.reshape(1, dh))
    a2 = _sc_scatter_sum(u, src, dst, zeros_h)

    pad = 128 - dout
    w2p = jnp.pad(W2, ((0, 0), (0, pad)))
    b2p = jnp.pad(b2, (0, pad)).reshape(1, 128)
    outp = _tc_out(a2, u, dinv, w2p, b2p)
    return outp[:, :dout]


# speedup vs baseline: 12.2634x; 12.2634x over previous
import functools

import jax
import jax.numpy as jnp
from jax import lax
from jax.experimental import pallas as pl
from jax.experimental.pallas import tpu as pltpu
from jax.experimental.pallas import tpu_sc as plsc

_NC = 2
_NS = 16
_NW = _NC * _NS
_C = 80


def _sc_scatter_sum(table, src, dst, zeros):
    n, d = table.shape
    e = src.shape[0]
    assert e % _NW == 0
    epw = e // _NW
    assert epw % _C == 0
    nchunk = epw // _C
    rpt = (n // _NS) // 8 * 8
    rem = n - _NS * rpt
    assert rem % 8 == 0

    mesh = plsc.VectorSubcoreMesh(core_axis_name="c", subcore_axis_name="s")

    @functools.partial(
        pl.kernel,
        mesh=mesh,
        out_type=jax.ShapeDtypeStruct((_NC * n, d), jnp.float32),
        scratch_types=[
            pltpu.VMEM((_C,), jnp.int32),
            pltpu.VMEM((_C,), jnp.int32),
            pltpu.VMEM((_C, d), jnp.float32),
            pltpu.VMEM_SHARED((n, d), jnp.float32),
            pltpu.SemaphoreType.DMA,
        ],
        compiler_params=pltpu.CompilerParams(use_tc_tiling_on_sc=False),
    )
    def k(table_hbm, src_hbm, dst_hbm, zeros_hbm, out_hbm,
          srcv, dstv, rows, acc, sem):
        c = lax.axis_index("c")
        s = lax.axis_index("s")
        w = c * _NS + s

        pltpu.sync_copy(zeros_hbm.at[pl.ds(s * rpt, rpt)],
                        acc.at[pl.ds(s * rpt, rpt)])
        if rem:
            @pl.when(s == _NS - 1)
            def _():
                pltpu.sync_copy(zeros_hbm.at[pl.ds(_NS * rpt, rem)],
                                acc.at[pl.ds(_NS * rpt, rem)])
        plsc.subcore_barrier()

        @pl.loop(0, nchunk)
        def _(i):
            base = w * epw + i * _C
            pltpu.sync_copy(src_hbm.at[pl.ds(base, _C)], srcv)
            pltpu.sync_copy(dst_hbm.at[pl.ds(base, _C)], dstv)
            pltpu.async_copy(table_hbm.at[srcv], rows, sem).wait()
            pltpu.sync_copy(rows, acc.at[dstv], add=True)

        plsc.subcore_barrier()
        pltpu.sync_copy(acc.at[pl.ds(s * rpt, rpt)],
                        out_hbm.at[pl.ds(c * n + s * rpt, rpt)])
        if rem:
            @pl.when(s == _NS - 1)
            def _():
                pltpu.sync_copy(acc.at[pl.ds(_NS * rpt, rem)],
                                out_hbm.at[pl.ds(c * n + _NS * rpt, rem)])

    return k(table, src, dst, zeros)


def _tc_layer1(x, w1, d0, d1):
    n = x.shape[0]
    dh = w1.shape[1]

    def body(x_ref, w_ref, d0_ref, d1_ref, y_ref, dinv_ref):
        deg = d0_ref[...] + d1_ref[...] + 1.0
        dinv = lax.rsqrt(deg)
        dinv_ref[...] = dinv
        xw = jnp.dot(x_ref[...], w_ref[...],
                     preferred_element_type=jnp.float32)
        y_ref[...] = xw * dinv

    return pl.pallas_call(
        body,
        out_shape=(
            jax.ShapeDtypeStruct((n, dh), jnp.float32),
            jax.ShapeDtypeStruct((n, 1), jnp.float32),
        ),
    )(x, w1, d0, d1)


def _tc_mid(a, y1, dinv, b1):
    n, dh = y1.shape

    def body(a_ref, y_ref, dinv_ref, b_ref, u_ref):
        total = a_ref[:n] + a_ref[n:] + y_ref[...]
        h = jnp.maximum(dinv_ref[...] * total + b_ref[...], 0.0)
        u_ref[...] = dinv_ref[...] * h

    return pl.pallas_call(
        body,
        out_shape=jax.ShapeDtypeStruct((n, dh), jnp.float32),
    )(a, y1, dinv, b1)


def _tc_out(a, u, dinv, w2p, b2p):
    n, dh = u.shape
    dp = w2p.shape[1]

    def body(a_ref, u_ref, dinv_ref, w_ref, b_ref, o_ref):
        m = dinv_ref[...] * (a_ref[:n] + a_ref[n:] + u_ref[...])
        o_ref[...] = jnp.dot(m, w_ref[...],
                             preferred_element_type=jnp.float32) + b_ref[...]

    return pl.pallas_call(
        body,
        out_shape=jax.ShapeDtypeStruct((n, dp), jnp.float32),
    )(a, u, dinv, w2p, b2p)


def kernel(x, edge_index, W1, b1, W2, b2):
    n = x.shape[0]
    dh = W1.shape[1]
    dout = W2.shape[1]
    src = edge_index[0]
    dst = edge_index[1]

    zeros_h = jnp.zeros((n, dh), jnp.float32)
    zeros_d = jnp.zeros((n, 16), jnp.float32)
    ones_d = jnp.ones((n, 16), jnp.float32)

    degp = _sc_scatter_sum(ones_d, src, dst, zeros_d)
    d0 = degp[:n, 0:1]
    d1 = degp[n:, 0:1]

    y1, dinv = _tc_layer1(x, W1, d0, d1)
    a1 = _sc_scatter_sum(y1, src, dst, zeros_h)
    u = _tc_mid(a1, y1, dinv, b1.reshape(1, dh))
    a2 = _sc_scatter_sum(u, src, dst, zeros_h)

    pad = 128 - dout
    w2p = jnp.pad(W2, ((0, 0), (0, pad)))
    b2p = jnp.pad(b2, (0, pad)).reshape(1, 128)
    outp = _tc_out(a2, u, dinv, w2p, b2p)
    return outp[:, :dout]


# speedup vs baseline: 34.1593x; 2.7855x over previous
import functools

import jax
import jax.numpy as jnp
from jax import lax
from jax.experimental import pallas as pl
from jax.experimental.pallas import tpu as pltpu
from jax.experimental.pallas import tpu_sc as plsc

_NC = 2
_NS = 16
_NW = _NC * _NS
_C = 80


def _acc_init_and_writeout(n):
    rpt = (n // _NS) // 8 * 8
    rem = n - _NS * rpt
    assert rem % 8 == 0
    return rpt, rem


def _zero_acc(zeros_hbm, acc, s, rpt, rem):
    pltpu.sync_copy(zeros_hbm.at[pl.ds(s * rpt, rpt)],
                    acc.at[pl.ds(s * rpt, rpt)])
    if rem:
        @pl.when(s == _NS - 1)
        def _():
            pltpu.sync_copy(zeros_hbm.at[pl.ds(_NS * rpt, rem)],
                            acc.at[pl.ds(_NS * rpt, rem)])


def _write_acc(acc, out_hbm, c, s, n, rpt, rem):
    pltpu.sync_copy(acc.at[pl.ds(s * rpt, rpt)],
                    out_hbm.at[pl.ds(c * n + s * rpt, rpt)])
    if rem:
        @pl.when(s == _NS - 1)
        def _():
            pltpu.sync_copy(acc.at[pl.ds(_NS * rpt, rem)],
                            out_hbm.at[pl.ds(c * n + _NS * rpt, rem)])


def _sc_scatter_sum(table, src, dst, zeros):
    n, d = table.shape
    e = src.shape[0]
    assert e % _NW == 0
    epw = e // _NW
    assert epw % _C == 0
    nchunk = epw // _C
    half = nchunk // 2
    odd = nchunk % 2
    rpt, rem = _acc_init_and_writeout(n)

    mesh = plsc.VectorSubcoreMesh(core_axis_name="c", subcore_axis_name="s")

    @functools.partial(
        pl.kernel,
        mesh=mesh,
        out_type=jax.ShapeDtypeStruct((_NC * n, d), jnp.float32),
        scratch_types=[
            pltpu.VMEM((epw,), jnp.int32),
            pltpu.VMEM((epw,), jnp.int32),
            pltpu.VMEM((_C, d), jnp.float32),
            pltpu.VMEM((_C, d), jnp.float32),
            pltpu.VMEM_SHARED((n, d), jnp.float32),
            pltpu.SemaphoreType.DMA,
            pltpu.SemaphoreType.DMA,
        ],
        compiler_params=pltpu.CompilerParams(use_tc_tiling_on_sc=False),
    )
    def k(table_hbm, src_hbm, dst_hbm, zeros_hbm, out_hbm,
          srcs, dsts, rows0, rows1, acc, sem0, sem1):
        c = lax.axis_index("c")
        s = lax.axis_index("s")
        w = c * _NS + s

        pltpu.sync_copy(src_hbm.at[pl.ds(w * epw, epw)], srcs)
        pltpu.sync_copy(dst_hbm.at[pl.ds(w * epw, epw)], dsts)
        _zero_acc(zeros_hbm, acc, s, rpt, rem)
        plsc.subcore_barrier()

        def gather(i, rows, sem):
            return pltpu.async_copy(
                table_hbm.at[srcs.at[pl.ds(i * _C, _C)]], rows, sem)

        def scatter(i, rows):
            pltpu.sync_copy(rows, acc.at[dsts.at[pl.ds(i * _C, _C)]],
                            add=True)

        g0 = gather(0, rows0, sem0)

        @pl.loop(0, half)
        def _(kk):
            i0 = 2 * kk
            gather(i0 + 1, rows1, sem1)
            pltpu.make_async_copy(
                table_hbm.at[srcs.at[pl.ds(i0 * _C, _C)]], rows0, sem0
            ).wait()
            scatter(i0, rows0)

            @pl.when(i0 + 2 < nchunk)
            def _():
                gather(i0 + 2, rows0, sem0)

            pltpu.make_async_copy(
                table_hbm.at[srcs.at[pl.ds((i0 + 1) * _C, _C)]], rows1, sem1
            ).wait()
            scatter(i0 + 1, rows1)

        if odd:
            last = nchunk - 1
            pltpu.make_async_copy(
                table_hbm.at[srcs.at[pl.ds(last * _C, _C)]], rows0, sem0
            ).wait()
            scatter(last, rows0)

        plsc.subcore_barrier()
        _write_acc(acc, out_hbm, c, s, n, rpt, rem)

    return k(table, src, dst, zeros)


def _sc_degree(dst, zeros, e_total):
    n = zeros.shape[0]
    d = zeros.shape[1]
    e = e_total
    assert e % _NW == 0
    epw = e // _NW
    assert epw % _C == 0
    nchunk = epw // _C
    rpt, rem = _acc_init_and_writeout(n)

    mesh = plsc.VectorSubcoreMesh(core_axis_name="c", subcore_axis_name="s")

    @functools.partial(
        pl.kernel,
        mesh=mesh,
        out_type=jax.ShapeDtypeStruct((_NC * n, d), jnp.float32),
        scratch_types=[
            pltpu.VMEM((epw,), jnp.int32),
            pltpu.VMEM((_C, d), jnp.float32),
            pltpu.VMEM_SHARED((n, d), jnp.float32),
        ],
        compiler_params=pltpu.CompilerParams(use_tc_tiling_on_sc=False),
    )
    def k(dst_hbm, zeros_hbm, out_hbm, dsts, ones, acc):
        c = lax.axis_index("c")
        s = lax.axis_index("s")
        w = c * _NS + s

        pltpu.sync_copy(dst_hbm.at[pl.ds(w * epw, epw)], dsts)

        @pl.loop(0, _C)
        def _(j):
            ones[j] = jnp.full((d,), 1.0, jnp.float32)

        _zero_acc(zeros_hbm, acc, s, rpt, rem)
        plsc.subcore_barrier()

        @pl.loop(0, nchunk)
        def _(i):
            pltpu.sync_copy(ones, acc.at[dsts.at[pl.ds(i * _C, _C)]],
                            add=True)

        plsc.subcore_barrier()
        _write_acc(acc, out_hbm, c, s, n, rpt, rem)

    return k(dst, zeros)


def _tc_layer1(x, w1, d0, d1):
    n = x.shape[0]
    dh = w1.shape[1]

    def body(x_ref, w_ref, d0_ref, d1_ref, y_ref, dinv_ref):
        deg = d0_ref[...] + d1_ref[...] + 1.0
        dinv = lax.rsqrt(deg)
        dinv_ref[...] = dinv
        xw = jnp.dot(x_ref[...], w_ref[...],
                     preferred_element_type=jnp.float32)
        y_ref[...] = xw * dinv

    return pl.pallas_call(
        body,
        out_shape=(
            jax.ShapeDtypeStruct((n, dh), jnp.float32),
            jax.ShapeDtypeStruct((n, 1), jnp.float32),
        ),
    )(x, w1, d0, d1)


def _tc_mid(a, y1, dinv, b1):
    n, dh = y1.shape

    def body(a_ref, y_ref, dinv_ref, b_ref, u_ref):
        total = a_ref[:n] + a_ref[n:] + y_ref[...]
        h = jnp.maximum(dinv_ref[...] * total + b_ref[...], 0.0)
        u_ref[...] = dinv_ref[...] * h

    return pl.pallas_call(
        body,
        out_shape=jax.ShapeDtypeStruct((n, dh), jnp.float32),
    )(a, y1, dinv, b1)


def _tc_out(a, u, dinv, w2p, b2p):
    n, dh = u.shape
    dp = w2p.shape[1]

    def body(a_ref, u_ref, dinv_ref, w_ref, b_ref, o_ref):
        m = dinv_ref[...] * (a_ref[:n] + a_ref[n:] + u_ref[...])
        o_ref[...] = jnp.dot(m, w_ref[...],
                             preferred_element_type=jnp.float32) + b_ref[...]

    return pl.pallas_call(
        body,
        out_shape=jax.ShapeDtypeStruct((n, dp), jnp.float32),
    )(a, u, dinv, w2p, b2p)


def kernel(x, edge_index, W1, b1, W2, b2):
    n = x.shape[0]
    dh = W1.shape[1]
    dout = W2.shape[1]
    src = edge_index[0]
    dst = edge_index[1]

    zeros_h = jnp.zeros((n, dh), jnp.float32)
    zeros_d = jnp.zeros((n, 16), jnp.float32)

    degp = _sc_degree(dst, zeros_d, src.shape[0])
    d0 = degp[:n, 0:1]
    d1 = degp[n:, 0:1]

    y1, dinv = _tc_layer1(x, W1, d0, d1)
    a1 = _sc_scatter_sum(y1, src, dst, zeros_h)
    u = _tc_mid(a1, y1, dinv, b1.reshape(1, dh))
    a2 = _sc_scatter_sum(u, src, dst, zeros_h)

    pad = 128 - dout
    w2p = jnp.pad(W2, ((0, 0), (0, pad)))
    b2p = jnp.pad(b2, (0, pad)).reshape(1, 128)
    outp = _tc_out(a2, u, dinv, w2p, b2p)
    return outp[:, :dout]


# speedup vs baseline: 38.9503x; 1.1403x over previous
import functools

import jax
import jax.numpy as jnp
from jax import lax
from jax.experimental import pallas as pl
from jax.experimental.pallas import tpu as pltpu
from jax.experimental.pallas import tpu_sc as plsc

_NC = 2
_NS = 16
_NW = _NC * _NS
_C = 80


def _acc_init_and_writeout(n):
    rpt = (n // _NS) // 8 * 8
    rem = n - _NS * rpt
    assert rem % 8 == 0
    return rpt, rem


def _zero_acc(zeros_hbm, acc, s, rpt, rem):
    pltpu.sync_copy(zeros_hbm.at[pl.ds(s * rpt, rpt)],
                    acc.at[pl.ds(s * rpt, rpt)])
    if rem:
        @pl.when(s == _NS - 1)
        def _():
            pltpu.sync_copy(zeros_hbm.at[pl.ds(_NS * rpt, rem)],
                            acc.at[pl.ds(_NS * rpt, rem)])


def _write_acc(acc, out_hbm, c, s, n, rpt, rem):
    pltpu.sync_copy(acc.at[pl.ds(s * rpt, rpt)],
                    out_hbm.at[pl.ds(c * n + s * rpt, rpt)])
    if rem:
        @pl.when(s == _NS - 1)
        def _():
            pltpu.sync_copy(acc.at[pl.ds(_NS * rpt, rem)],
                            out_hbm.at[pl.ds(c * n + _NS * rpt, rem)])


def _sc_scatter_sum(table, src, dst, zeros):
    n, d = table.shape
    e = src.shape[0]
    assert e % _NW == 0
    epw = e // _NW
    assert epw % _C == 0
    nchunk = epw // _C
    nb = 4
    tail = nchunk % nb
    groups = nchunk // nb
    assert nchunk >= nb
    rpt, rem = _acc_init_and_writeout(n)

    mesh = plsc.VectorSubcoreMesh(core_axis_name="c", subcore_axis_name="s")

    @functools.partial(
        pl.kernel,
        mesh=mesh,
        out_type=jax.ShapeDtypeStruct((_NC * n, d), jnp.float32),
        scratch_types=[
            pltpu.VMEM((epw,), jnp.int32),
            pltpu.VMEM((epw,), jnp.int32),
            [pltpu.VMEM((_C, d), jnp.float32)] * nb,
            [pltpu.SemaphoreType.DMA] * nb,
            [pltpu.SemaphoreType.DMA] * nb,
            pltpu.VMEM_SHARED((n, d), jnp.float32),
        ],
        compiler_params=pltpu.CompilerParams(use_tc_tiling_on_sc=False),
    )
    def k(table_hbm, src_hbm, dst_hbm, zeros_hbm, out_hbm,
          srcs, dsts, rows, gsem, ssem, acc):
        c = lax.axis_index("c")
        s = lax.axis_index("s")
        w = c * _NS + s

        pltpu.sync_copy(src_hbm.at[pl.ds(w * epw, epw)], srcs)
        pltpu.sync_copy(dst_hbm.at[pl.ds(w * epw, epw)], dsts)
        _zero_acc(zeros_hbm, acc, s, rpt, rem)
        plsc.subcore_barrier()

        def fire_gather(i, b):
            pltpu.async_copy(
                table_hbm.at[srcs.at[pl.ds(i * _C, _C)]], rows[b], gsem[b])

        def wait_gather(i, b):
            pltpu.make_async_copy(
                table_hbm.at[srcs.at[pl.ds(i * _C, _C)]], rows[b], gsem[b]
            ).wait()

        def fire_scatter(i, b):
            pltpu.async_copy(
                rows[b], acc.at[dsts.at[pl.ds(i * _C, _C)]], ssem[b],
                add=True)

        def wait_scatter(i, b):
            pltpu.make_async_copy(
                rows[b], acc.at[dsts.at[pl.ds(i * _C, _C)]], ssem[b]
            ).wait()

        for b in range(nb):
            fire_gather(b, b)

        @pl.loop(0, groups)
        def _(g):
            i0 = g * nb
            for b in range(nb):
                i = i0 + b
                wait_gather(i, b)
                fire_scatter(i, b)
            for b in range(nb):
                i = i0 + b

                @pl.when(i + nb < nchunk)
                def _():
                    wait_scatter(i, b)
                    fire_gather(i + nb, b)

        for b in range(tail):
            i = groups * nb + b
            wait_gather(i, b)
            fire_scatter(i, b)
        for b in range(nb):
            i = groups * nb + b if b < tail else (groups - 1) * nb + b
            wait_scatter(i, b)

        plsc.subcore_barrier()
        _write_acc(acc, out_hbm, c, s, n, rpt, rem)

    return k(table, src, dst, zeros)


def _sc_degree(dst, zeros, e_total):
    n = zeros.shape[0]
    d = zeros.shape[1]
    e = e_total
    assert e % _NW == 0
    epw = e // _NW
    assert epw % _C == 0
    nchunk = epw // _C
    rpt, rem = _acc_init_and_writeout(n)

    mesh = plsc.VectorSubcoreMesh(core_axis_name="c", subcore_axis_name="s")

    @functools.partial(
        pl.kernel,
        mesh=mesh,
        out_type=jax.ShapeDtypeStruct((_NC * n, d), jnp.float32),
        scratch_types=[
            pltpu.VMEM((epw,), jnp.int32),
            pltpu.VMEM((_C, d), jnp.float32),
            pltpu.VMEM_SHARED((n, d), jnp.float32),
        ],
        compiler_params=pltpu.CompilerParams(use_tc_tiling_on_sc=False),
    )
    def k(dst_hbm, zeros_hbm, out_hbm, dsts, ones, acc):
        c = lax.axis_index("c")
        s = lax.axis_index("s")
        w = c * _NS + s

        pltpu.sync_copy(dst_hbm.at[pl.ds(w * epw, epw)], dsts)

        @pl.loop(0, _C)
        def _(j):
            ones[j] = jnp.full((d,), 1.0, jnp.float32)

        _zero_acc(zeros_hbm, acc, s, rpt, rem)
        plsc.subcore_barrier()

        @pl.loop(0, nchunk)
        def _(i):
            pltpu.sync_copy(ones, acc.at[dsts.at[pl.ds(i * _C, _C)]],
                            add=True)

        plsc.subcore_barrier()
        _write_acc(acc, out_hbm, c, s, n, rpt, rem)

    return k(dst, zeros)


def _tc_layer1(x, w1, d0, d1):
    n = x.shape[0]
    dh = w1.shape[1]

    def body(x_ref, w_ref, d0_ref, d1_ref, y_ref, dinv_ref):
        deg = d0_ref[...] + d1_ref[...] + 1.0
        dinv = lax.rsqrt(deg)
        dinv_ref[...] = dinv
        xw = jnp.dot(x_ref[...], w_ref[...],
                     preferred_element_type=jnp.float32)
        y_ref[...] = xw * dinv

    return pl.pallas_call(
        body,
        out_shape=(
            jax.ShapeDtypeStruct((n, dh), jnp.float32),
            jax.ShapeDtypeStruct((n, 1), jnp.float32),
        ),
    )(x, w1, d0, d1)


def _tc_mid(a, y1, dinv, b1):
    n, dh = y1.shape

    def body(a_ref, y_ref, dinv_ref, b_ref, u_ref):
        total = a_ref[:n] + a_ref[n:] + y_ref[...]
        h = jnp.maximum(dinv_ref[...] * total + b_ref[...], 0.0)
        u_ref[...] = dinv_ref[...] * h

    return pl.pallas_call(
        body,
        out_shape=jax.ShapeDtypeStruct((n, dh), jnp.float32),
    )(a, y1, dinv, b1)


def _tc_out(a, u, dinv, w2p, b2p):
    n, dh = u.shape
    dp = w2p.shape[1]

    def body(a_ref, u_ref, dinv_ref, w_ref, b_ref, o_ref):
        m = dinv_ref[...] * (a_ref[:n] + a_ref[n:] + u_ref[...])
        o_ref[...] = jnp.dot(m, w_ref[...],
                             preferred_element_type=jnp.float32) + b_ref[...]

    return pl.pallas_call(
        body,
        out_shape=jax.ShapeDtypeStruct((n, dp), jnp.float32),
    )(a, u, dinv, w2p, b2p)


def kernel(x, edge_index, W1, b1, W2, b2):
    n = x.shape[0]
    dh = W1.shape[1]
    dout = W2.shape[1]
    src = edge_index[0]
    dst = edge_index[1]

    zeros_h = jnp.zeros((n, dh), jnp.float32)
    zeros_d = jnp.zeros((n, 16), jnp.float32)

    degp = _sc_degree(dst, zeros_d, src.shape[0])
    d0 = degp[:n, 0:1]
    d1 = degp[n:, 0:1]

    y1, dinv = _tc_layer1(x, W1, d0, d1)
    a1 = _sc_scatter_sum(y1, src, dst, zeros_h)
    u = _tc_mid(a1, y1, dinv, b1.reshape(1, dh))
    a2 = _sc_scatter_sum(u, src, dst, zeros_h)

    pad = 128 - dout
    w2p = jnp.pad(W2, ((0, 0), (0, pad)))
    b2p = jnp.pad(b2, (0, pad)).reshape(1, 128)
    outp = _tc_out(a2, u, dinv, w2p, b2p)
    return outp[:, :dout]


# speedup vs baseline: 42.7112x; 1.0966x over previous
import functools

import jax
import jax.numpy as jnp
from jax import lax
from jax.experimental import pallas as pl
from jax.experimental.pallas import tpu as pltpu
from jax.experimental.pallas import tpu_sc as plsc

_NC = 2
_NS = 16
_NW = _NC * _NS
_C = 80
_NB = 4


def _acc_slabs(n):
    rpt = (n // _NS) // 8 * 8
    rem = n - _NS * rpt
    assert rem % 8 == 0
    return rpt, rem


def _zero_acc(zeros_hbm, acc, s, rpt, rem):
    pltpu.sync_copy(zeros_hbm.at[pl.ds(0, rpt)],
                    acc.at[pl.ds(s * rpt, rpt)])
    if rem:
        @pl.when(s == _NS - 1)
        def _():
            pltpu.sync_copy(zeros_hbm.at[pl.ds(0, rem)],
                            acc.at[pl.ds(_NS * rpt, rem)])


def _write_acc(acc, out_hbm, c, s, n, rpt, rem):
    pltpu.sync_copy(acc.at[pl.ds(s * rpt, rpt)],
                    out_hbm.at[pl.ds(c * n + s * rpt, rpt)])
    if rem:
        @pl.when(s == _NS - 1)
        def _():
            pltpu.sync_copy(acc.at[pl.ds(_NS * rpt, rem)],
                            out_hbm.at[pl.ds(c * n + _NS * rpt, rem)])


def _sc_scatter_sum(table, edge_index, zeros):
    n, d = table.shape
    e = edge_index.shape[1]
    assert e % _NW == 0
    epw = e // _NW
    assert epw % _C == 0
    nchunk = epw // _C
    tail = nchunk % _NB
    groups = nchunk // _NB
    assert nchunk >= _NB
    rpt, rem = _acc_slabs(n)

    mesh = plsc.VectorSubcoreMesh(core_axis_name="c", subcore_axis_name="s")

    @functools.partial(
        pl.kernel,
        mesh=mesh,
        out_type=jax.ShapeDtypeStruct((_NC * n, d), jnp.float32),
        scratch_types=[
            pltpu.VMEM((epw,), jnp.int32),
            pltpu.VMEM((epw,), jnp.int32),
            [pltpu.VMEM((_C, d), jnp.float32)] * _NB,
            [pltpu.SemaphoreType.DMA] * _NB,
            [pltpu.SemaphoreType.DMA] * _NB,
            pltpu.VMEM_SHARED((n, d), jnp.float32),
        ],
        compiler_params=pltpu.CompilerParams(use_tc_tiling_on_sc=False),
    )
    def k(table_hbm, ei_hbm, zeros_hbm, out_hbm,
          srcs, dsts, rows, gsem, ssem, acc):
        c = lax.axis_index("c")
        s = lax.axis_index("s")
        w = c * _NS + s

        pltpu.sync_copy(ei_hbm.at[0].at[pl.ds(w * epw, epw)], srcs)
        pltpu.sync_copy(ei_hbm.at[1].at[pl.ds(w * epw, epw)], dsts)
        _zero_acc(zeros_hbm, acc, s, rpt, rem)
        plsc.subcore_barrier()

        def fire_gather(i, b):
            pltpu.async_copy(
                table_hbm.at[srcs.at[pl.ds(i * _C, _C)]], rows[b], gsem[b])

        def wait_gather(i, b):
            pltpu.make_async_copy(
                table_hbm.at[srcs.at[pl.ds(i * _C, _C)]], rows[b], gsem[b]
            ).wait()

        def fire_scatter(i, b):
            pltpu.async_copy(
                rows[b], acc.at[dsts.at[pl.ds(i * _C, _C)]], ssem[b],
                add=True)

        def wait_scatter(i, b):
            pltpu.make_async_copy(
                rows[b], acc.at[dsts.at[pl.ds(i * _C, _C)]], ssem[b]
            ).wait()

        for b in range(_NB):
            fire_gather(b, b)

        @pl.loop(0, groups)
        def _(g):
            i0 = g * _NB
            for b in range(_NB):
                i = i0 + b
                wait_gather(i, b)
                fire_scatter(i, b)
            for b in range(_NB):
                i = i0 + b

                @pl.when(i + _NB < nchunk)
                def _():
                    wait_scatter(i, b)
                    fire_gather(i + _NB, b)

        for b in range(tail):
            i = groups * _NB + b
            wait_gather(i, b)
            fire_scatter(i, b)
        for b in range(_NB):
            i = groups * _NB + b if b < tail else (groups - 1) * _NB + b
            wait_scatter(i, b)

        plsc.subcore_barrier()
        _write_acc(acc, out_hbm, c, s, n, rpt, rem)

    return k(table, edge_index, zeros)


def _sc_degree(edge_index, zeros, n):
    d = zeros.shape[1]
    e = edge_index.shape[1]
    assert e % _NW == 0
    epw = e // _NW
    assert epw % _C == 0
    nchunk = epw // _C
    rpt, rem = _acc_slabs(n)

    mesh = plsc.VectorSubcoreMesh(core_axis_name="c", subcore_axis_name="s")

    @functools.partial(
        pl.kernel,
        mesh=mesh,
        out_type=jax.ShapeDtypeStruct((_NC * n, d), jnp.float32),
        scratch_types=[
            pltpu.VMEM((epw,), jnp.int32),
            pltpu.VMEM((_C, d), jnp.float32),
            pltpu.VMEM_SHARED((n, d), jnp.float32),
            pltpu.SemaphoreType.DMA,
        ],
        compiler_params=pltpu.CompilerParams(use_tc_tiling_on_sc=False),
    )
    def k(ei_hbm, zeros_hbm, out_hbm, dsts, ones, acc, ssem):
        c = lax.axis_index("c")
        s = lax.axis_index("s")
        w = c * _NS + s

        pltpu.sync_copy(ei_hbm.at[1].at[pl.ds(w * epw, epw)], dsts)

        @pl.loop(0, _C)
        def _(j):
            ones[j] = jnp.full((d,), 1.0, jnp.float32)

        _zero_acc(zeros_hbm, acc, s, rpt, rem)
        plsc.subcore_barrier()

        @pl.loop(0, nchunk)
        def _(i):
            pltpu.async_copy(ones, acc.at[dsts.at[pl.ds(i * _C, _C)]],
                             ssem, add=True)

        @pl.loop(0, nchunk)
        def _(i):
            pltpu.make_async_copy(
                ones, acc.at[dsts.at[pl.ds(i * _C, _C)]], ssem).wait()

        plsc.subcore_barrier()
        _write_acc(acc, out_hbm, c, s, n, rpt, rem)

    return k(edge_index, zeros)


_BLK = 2000


def _tc_layer1(x, w1, degp):
    n, din = x.shape
    dh = w1.shape[1]
    dd = degp.shape[2]

    def body(x_ref, w_ref, dp_ref, y_ref, dinv_ref):
        deg = dp_ref[0, :, 0:1] + dp_ref[1, :, 0:1] + 1.0
        dinv = lax.rsqrt(deg)
        dinv_ref[...] = dinv
        xw = jnp.dot(x_ref[...], w_ref[...],
                     preferred_element_type=jnp.float32)
        y_ref[...] = xw * dinv

    grid = n // _BLK
    return pl.pallas_call(
        body,
        grid=(grid,),
        in_specs=[
            pl.BlockSpec((_BLK, din), lambda i: (i, 0)),
            pl.BlockSpec((din, dh), lambda i: (0, 0)),
            pl.BlockSpec((_NC, _BLK, dd), lambda i: (0, i, 0)),
        ],
        out_specs=(
            pl.BlockSpec((_BLK, dh), lambda i: (i, 0)),
            pl.BlockSpec((_BLK, 1), lambda i: (i, 0)),
        ),
        out_shape=(
            jax.ShapeDtypeStruct((n, dh), jnp.float32),
            jax.ShapeDtypeStruct((n, 1), jnp.float32),
        ),
    )(x, w1, degp)


def _tc_mid(a, y1, dinv, b1):
    n, dh = y1.shape

    def body(a_ref, y_ref, dinv_ref, b_ref, u_ref):
        total = a_ref[0] + a_ref[1] + y_ref[...]
        h = jnp.maximum(dinv_ref[...] * total + b_ref[...], 0.0)
        u_ref[...] = dinv_ref[...] * h

    grid = n // _BLK
    return pl.pallas_call(
        body,
        grid=(grid,),
        in_specs=[
            pl.BlockSpec((_NC, _BLK, dh), lambda i: (0, i, 0)),
            pl.BlockSpec((_BLK, dh), lambda i: (i, 0)),
            pl.BlockSpec((_BLK, 1), lambda i: (i, 0)),
            pl.BlockSpec((1, dh), lambda i: (0, 0)),
        ],
        out_specs=pl.BlockSpec((_BLK, dh), lambda i: (i, 0)),
        out_shape=jax.ShapeDtypeStruct((n, dh), jnp.float32),
    )(a, y1, dinv, b1)


def _tc_out(a, u, dinv, w2, b2):
    n, dh = u.shape
    dout = w2.shape[1]

    def body(a_ref, u_ref, dinv_ref, w_ref, b_ref, o_ref):
        m = dinv_ref[...] * (a_ref[0] + a_ref[1] + u_ref[...])
        o_ref[...] = jnp.dot(m, w_ref[...],
                             preferred_element_type=jnp.float32) + b_ref[...]

    grid = n // _BLK
    return pl.pallas_call(
        body,
        grid=(grid,),
        in_specs=[
            pl.BlockSpec((_NC, _BLK, dh), lambda i: (0, i, 0)),
            pl.BlockSpec((_BLK, dh), lambda i: (i, 0)),
            pl.BlockSpec((_BLK, 1), lambda i: (i, 0)),
            pl.BlockSpec((dh, dout), lambda i: (0, 0)),
            pl.BlockSpec((1, dout), lambda i: (0, 0)),
        ],
        out_specs=pl.BlockSpec((_BLK, dout), lambda i: (i, 0)),
        out_shape=jax.ShapeDtypeStruct((n, dout), jnp.float32),
    )(a, u, dinv, w2, b2)


def kernel(x, edge_index, W1, b1, W2, b2):
    n = x.shape[0]
    dh = W1.shape[1]

    rpt, rem = _acc_slabs(n)
    zrows = max(rpt, rem)
    zeros_h = jnp.zeros((zrows, dh), jnp.float32)
    zeros_d = jnp.zeros((zrows, 16), jnp.float32)

    degp = _sc_degree(edge_index, zeros_d, n).reshape(_NC, n, 16)

    y1, dinv = _tc_layer1(x, W1, degp)
    a1 = _sc_scatter_sum(y1, edge_index, zeros_h).reshape(_NC, n, dh)
    u = _tc_mid(a1, y1, dinv, b1.reshape(1, dh))
    a2 = _sc_scatter_sum(u, edge_index, zeros_h).reshape(_NC, n, dh)
    return _tc_out(a2, u, dinv, W2, b2.reshape(1, W2.shape[1]))
